# baseline (speedup 1.0000x reference)
import functools

import jax
import jax.numpy as jnp
import numpy as np
from jax import lax
from jax.experimental import pallas as pl
from jax.experimental.pallas import tpu as pltpu
from jax.experimental.pallas import tpu_sc as plsc

E = 8
K = 2
D_MODEL = 1024
D_FF = 4096
CAP_FACTOR = 1.25

FBLK = 1024
NF = D_FF // FBLK
MROWS = 256

TBLK = 256


def _router_body(x_ref, wr_ref, slot0_ref, slot1_ref, w0_ref, w1_ref,
                 counts_ref, carry_ref, *, cap, nblk):
    b = pl.program_id(0)

    @pl.when(b == 0)
    def _init():
        carry_ref[...] = jnp.zeros_like(carry_ref)

    xb = x_ref[...].astype(jnp.bfloat16)
    wrb = wr_ref[...].astype(jnp.bfloat16)
    logits = jax.lax.dot_general(
        xb, wrb, (((1,), (1,)), ((), ())),
        preferred_element_type=jnp.float32)
    logits = jnp.clip(logits, -10000.0, 10000.0)
    m = jnp.max(logits, axis=1, keepdims=True)
    p = jnp.exp(logits - m)
    p = p / (jnp.sum(p, axis=1, keepdims=True) + 1e-12)

    iota_e = jax.lax.broadcasted_iota(jnp.int32, (TBLK, E), 1)
    p1 = jnp.max(p, axis=1, keepdims=True)
    a1 = jnp.min(jnp.where(p == p1, iota_e, E), axis=1, keepdims=True)
    masked = jnp.where(iota_e == a1, -1.0, p)
    p2 = jnp.max(masked, axis=1, keepdims=True)
    a2 = jnp.min(jnp.where(masked == p2, iota_e, E), axis=1, keepdims=True)

    lane = jax.lax.broadcasted_iota(jnp.int32, (TBLK, 128), 1)
    A0 = (lane == a1).astype(jnp.int32)
    A1 = (lane == a2).astype(jnp.int32)
    s = A0 + A1
    inc = s
    for sh in (1, 2, 4, 8, 16, 32, 64, 128):
        if sh < TBLK:
            inc = inc + jnp.pad(inc, ((sh, 0), (0, 0)))[:TBLK]
    base = carry_ref[0:1, :] + inc
    rank0 = jnp.sum((base - A0 - A1) * A0, axis=1, keepdims=True)
    rank1 = jnp.sum((base - A1) * A1, axis=1, keepdims=True)
    carry_ref[0:1, :] += jnp.sum(s, axis=0, keepdims=True)

    keep0 = rank0 < cap
    keep1 = rank1 < cap
    slot0 = jnp.where(keep0, a1 * cap + rank0, E * cap)
    slot1 = jnp.where(keep1, a2 * cap + rank1, E * cap)
    w0 = jnp.where(keep0, p1, 0.0)
    w1 = jnp.where(keep1, p2, 0.0)

    slot0_ref[...] = slot0.reshape(1, 1, TBLK)
    slot1_ref[...] = slot1.reshape(1, 1, TBLK)
    w0_ref[...] = w0.reshape(1, 1, TBLK)
    w1_ref[...] = w1.reshape(1, 1, TBLK)

    @pl.when(b == nblk - 1)
    def _fin():
        total = carry_ref[0:1, :] + jnp.zeros((E, 128), jnp.int32)
        counts_ref[...] = jnp.minimum(total, cap)


def _route(x_flat, Wr, cap):
    T = x_flat.shape[0]
    nblk = T // TBLK
    body = functools.partial(_router_body, cap=cap, nblk=nblk)
    out_shapes = [
        jax.ShapeDtypeStruct((nblk, 1, TBLK), jnp.int32),
        jax.ShapeDtypeStruct((nblk, 1, TBLK), jnp.int32),
        jax.ShapeDtypeStruct((nblk, 1, TBLK), jnp.float32),
        jax.ShapeDtypeStruct((nblk, 1, TBLK), jnp.float32),
        jax.ShapeDtypeStruct((E, 128), jnp.int32),
    ]
    blk = lambda b: (b, 0, 0)
    out_specs = [
        pl.BlockSpec((1, 1, TBLK), blk),
        pl.BlockSpec((1, 1, TBLK), blk),
        pl.BlockSpec((1, 1, TBLK), blk),
        pl.BlockSpec((1, 1, TBLK), blk),
        pl.BlockSpec((E, 128), lambda b: (0, 0)),
    ]
    slot0, slot1, w0, w1, counts = pl.pallas_call(
        body,
        grid=(nblk,),
        in_specs=[
            pl.BlockSpec((TBLK, D_MODEL), lambda b: (b, 0)),
            pl.BlockSpec((E, D_MODEL), lambda b: (0, 0)),
        ],
        out_specs=out_specs,
        out_shape=out_shapes,
        scratch_shapes=[pltpu.VMEM((8, 128), jnp.int32)],
    )(x_flat, Wr)
    return (slot0.reshape(T), slot1.reshape(T), w0.reshape(T),
            w1.reshape(T), counts[0, :E])


NW = 32
DISP_CHT = 64


def _sc_dispatch(x_flat, scat_idx, n_slots):
    T, D = x_flat.shape
    tpw = T // NW
    nch = tpw // DISP_CHT
    mesh = plsc.VectorSubcoreMesh(core_axis_name="c", subcore_axis_name="s")

    @functools.partial(
        pl.kernel, mesh=mesh,
        out_type=jax.ShapeDtypeStruct((n_slots, D), jnp.float32),
        scratch_types=[
            pltpu.VMEM((DISP_CHT, D), jnp.float32),
            pltpu.VMEM((DISP_CHT,), jnp.int32),
            pltpu.VMEM((DISP_CHT,), jnp.int32),
            pltpu.SemaphoreType.DMA,
        ],
    )
    def k(x_hbm, idx_hbm, xe_hbm, xbuf, i0, i1, sem):
        wid = lax.axis_index("s") * 2 + lax.axis_index("c")
        for ch in range(nch):
            tok0 = wid * tpw + ch * DISP_CHT
            pltpu.sync_copy(x_hbm.at[pl.ds(tok0, DISP_CHT)], xbuf)
            pltpu.sync_copy(idx_hbm.at[wid, 2 * ch + 0], i0)
            pltpu.sync_copy(idx_hbm.at[wid, 2 * ch + 1], i1)
            pltpu.async_copy(xbuf, xe_hbm.at[i0], sem).wait()
            pltpu.async_copy(xbuf, xe_hbm.at[i1], sem).wait()

    return k(x_flat, scat_idx)


COMB_CHT = 32


def _sc_combine(ye, comb_idx, wb):
    n_slots, D = ye.shape
    T = NW * comb_idx.shape[1] // 2 * COMB_CHT
    tpw = T // NW
    nch = tpw // COMB_CHT
    nl = D // 16
    mesh = plsc.VectorSubcoreMesh(core_axis_name="c", subcore_axis_name="s")

    @functools.partial(
        pl.kernel, mesh=mesh,
        out_type=jax.ShapeDtypeStruct((T, D), jnp.float32),
        scratch_types=[
            pltpu.VMEM((COMB_CHT, D), jnp.float32),
            pltpu.VMEM((COMB_CHT, D), jnp.float32),
            pltpu.VMEM((COMB_CHT, D), jnp.float32),
            pltpu.VMEM((COMB_CHT,), jnp.int32),
            pltpu.VMEM((COMB_CHT,), jnp.int32),
            pltpu.VMEM((COMB_CHT, 16), jnp.float32),
            pltpu.VMEM((COMB_CHT, 16), jnp.float32),
            pltpu.SemaphoreType.DMA,
            pltpu.SemaphoreType.DMA,
        ],
    )
    def k(ye_hbm, idx_hbm, wb_hbm, out_hbm,
          r0, r1, ob, i0, i1, wv0, wv1, sem0, sem1):
        wid = lax.axis_index("s") * 2 + lax.axis_index("c")
        for ch in range(nch):
            tok0 = wid * tpw + ch * COMB_CHT
            pltpu.sync_copy(idx_hbm.at[wid, 2 * ch + 0], i0)
            pltpu.sync_copy(idx_hbm.at[wid, 2 * ch + 1], i1)
            pltpu.sync_copy(wb_hbm.at[wid, 2 * ch + 0], wv0)
            pltpu.sync_copy(wb_hbm.at[wid, 2 * ch + 1], wv1)
            cp0 = pltpu.async_copy(ye_hbm.at[i0], r0, sem0)
            cp1 = pltpu.async_copy(ye_hbm.at[i1], r1, sem1)
            cp0.wait()
            cp1.wait()

            def body(j, carry):
                a = wv0[j]
                b = wv1[j]
                for l in range(nl):
                    ob[j, pl.ds(l * 16, 16)] = (
                        a * r0[j, pl.ds(l * 16, 16)]
                        + b * r1[j, pl.ds(l * 16, 16)])
                return carry

            lax.fori_loop(0, COMB_CHT, body, 0)
            pltpu.sync_copy(ob, out_hbm.at[pl.ds(tok0, COMB_CHT)])

    return k(ye, comb_idx, wb)


def _ffn_body(counts_ref, xe_ref, w1_ref, w2_ref, out_ref, acc_ref, xb_ref):
    f = pl.program_id(1)
    e = pl.program_id(0)

    @pl.when(f == 0)
    def _init():
        xb_ref[...] = xe_ref[...].astype(jnp.bfloat16)

    w1b = w1_ref[0].astype(jnp.bfloat16)
    w2b = w2_ref[0].astype(jnp.bfloat16)
    cnt = counts_ref[e]
    xb = xb_ref[...]
    h = jax.lax.dot_general(
        xb, w1b, (((1,), (1,)), ((), ())),
        preferred_element_type=jnp.float32)
    h = 0.5 * h * (1.0 + jax.lax.erf(h * np.float32(0.7071067811865476)))
    contrib = jax.lax.dot_general(
        h.astype(jnp.bfloat16), w2b, (((1,), (1,)), ((), ())),
        preferred_element_type=jnp.float32)

    @pl.when(f == 0)
    def _first():
        acc_ref[...] = contrib

    @pl.when(f != 0)
    def _rest():
        acc_ref[...] += contrib

    @pl.when(f == NF - 1)
    def _fin():
        rows = jax.lax.broadcasted_iota(jnp.int32, acc_ref.shape, 0)
        out_ref[...] = jnp.where(rows < cnt, acc_ref[...], 0.0)


def _expert_ffn(xe, W1, W2, counts, cap):
    return pl.pallas_call(
        _ffn_body,
        grid=(E, NF),
        in_specs=[
            pl.BlockSpec(memory_space=pltpu.SMEM),
            pl.BlockSpec((cap, D_MODEL), lambda e, f: (e, 0)),
            pl.BlockSpec((1, FBLK, D_MODEL), lambda e, f: (e, f, 0)),
            pl.BlockSpec((1, D_MODEL, FBLK), lambda e, f: (e, 0, f)),
        ],
        out_specs=pl.BlockSpec((cap, D_MODEL), lambda e, f: (e, 0)),
        out_shape=jax.ShapeDtypeStruct((E * cap, D_MODEL), jnp.float32),
        scratch_shapes=[
            pltpu.VMEM((cap, D_MODEL), jnp.float32),
            pltpu.VMEM((cap, D_MODEL), jnp.bfloat16),
        ],
    )(counts, xe, W1, W2)


def kernel(x, Wr, W1, W2):
    B, S, D = x.shape
    x_flat = x.reshape(-1, D)
    T = x_flat.shape[0]
    cap = max(1, int(np.ceil(T * K * CAP_FACTOR / E)))

    slot0, slot1, w0, w1, counts_k = _route(x_flat, Wr, cap)

    tpw = T // NW
    nch = tpw // DISP_CHT
    scat_idx = (
        jnp.stack([slot0, slot1], axis=1)
        .reshape(NW, nch, DISP_CHT, 2)
        .transpose(0, 1, 3, 2)
        .reshape(NW, nch * 2, DISP_CHT))
    xe_buf = _sc_dispatch(x_flat, scat_idx, E * cap + 8)

    ye = _expert_ffn(xe_buf, W1, W2, counts_k, cap)

    nchc = (T // NW) // COMB_CHT
    comb_idx = (
        jnp.minimum(jnp.stack([slot0, slot1], axis=1), E * cap - 1)
        .reshape(NW, nchc, COMB_CHT, 2)
        .transpose(0, 1, 3, 2)
        .reshape(NW, nchc * 2, COMB_CHT))
    wb = (
        jnp.stack([w0, w1], axis=1)
        .reshape(NW, nchc, COMB_CHT, 2)
        .transpose(0, 1, 3, 2)
        .reshape(NW, nchc * 2, COMB_CHT)[..., None]
        * jnp.ones((1, 1, 1, 16), jnp.float32))
    out = _sc_combine(ye, comb_idx, wb)
    return out.reshape(B, S, D)


# speedup vs baseline: 1.0579x; 1.0579x over previous
import functools

import jax
import jax.numpy as jnp
import numpy as np
from jax import lax
from jax.experimental import pallas as pl
from jax.experimental.pallas import tpu as pltpu
from jax.experimental.pallas import tpu_sc as plsc

E = 8
K = 2
D_MODEL = 1024
D_FF = 4096
CAP_FACTOR = 1.25

FBLK = 1024
NF = D_FF // FBLK
MROWS = 256

TBLK = 256


def _router_body(x_ref, wr_ref, slot0_ref, slot1_ref, w0_ref, w1_ref,
                 counts_ref, carry_ref, *, cap, nblk):
    b = pl.program_id(0)

    @pl.when(b == 0)
    def _init():
        carry_ref[...] = jnp.zeros_like(carry_ref)

    xb = x_ref[...].astype(jnp.bfloat16)
    wrb = wr_ref[...].astype(jnp.bfloat16)
    logits = jax.lax.dot_general(
        xb, wrb, (((1,), (1,)), ((), ())),
        preferred_element_type=jnp.float32)
    logits = jnp.clip(logits, -10000.0, 10000.0)
    m = jnp.max(logits, axis=1, keepdims=True)
    p = jnp.exp(logits - m)
    p = p / (jnp.sum(p, axis=1, keepdims=True) + 1e-12)

    iota_e = jax.lax.broadcasted_iota(jnp.int32, (TBLK, E), 1)
    p1 = jnp.max(p, axis=1, keepdims=True)
    a1 = jnp.min(jnp.where(p == p1, iota_e, E), axis=1, keepdims=True)
    masked = jnp.where(iota_e == a1, -1.0, p)
    p2 = jnp.max(masked, axis=1, keepdims=True)
    a2 = jnp.min(jnp.where(masked == p2, iota_e, E), axis=1, keepdims=True)

    lane = jax.lax.broadcasted_iota(jnp.int32, (TBLK, 128), 1)
    A0 = (lane == a1).astype(jnp.int32)
    A1 = (lane == a2).astype(jnp.int32)
    s = A0 + A1
    inc = s
    for sh in (1, 2, 4, 8, 16, 32, 64, 128):
        if sh < TBLK:
            inc = inc + jnp.pad(inc, ((sh, 0), (0, 0)))[:TBLK]
    base = carry_ref[0:1, :] + inc
    rank0 = jnp.sum((base - A0 - A1) * A0, axis=1, keepdims=True)
    rank1 = jnp.sum((base - A1) * A1, axis=1, keepdims=True)
    carry_ref[0:1, :] += jnp.sum(s, axis=0, keepdims=True)

    keep0 = rank0 < cap
    keep1 = rank1 < cap
    slot0 = jnp.where(keep0, a1 * cap + rank0, E * cap)
    slot1 = jnp.where(keep1, a2 * cap + rank1, E * cap)
    w0 = jnp.where(keep0, p1, 0.0)
    w1 = jnp.where(keep1, p2, 0.0)

    slot0_ref[...] = slot0.reshape(1, 1, TBLK)
    slot1_ref[...] = slot1.reshape(1, 1, TBLK)
    w0_ref[...] = w0.reshape(1, 1, TBLK)
    w1_ref[...] = w1.reshape(1, 1, TBLK)

    @pl.when(b == nblk - 1)
    def _fin():
        total = carry_ref[0:1, :] + jnp.zeros((E, 128), jnp.int32)
        counts_ref[...] = jnp.minimum(total, cap)


def _route(x_flat, Wr, cap):
    T = x_flat.shape[0]
    nblk = T // TBLK
    body = functools.partial(_router_body, cap=cap, nblk=nblk)
    out_shapes = [
        jax.ShapeDtypeStruct((nblk, 1, TBLK), jnp.int32),
        jax.ShapeDtypeStruct((nblk, 1, TBLK), jnp.int32),
        jax.ShapeDtypeStruct((nblk, 1, TBLK), jnp.float32),
        jax.ShapeDtypeStruct((nblk, 1, TBLK), jnp.float32),
        jax.ShapeDtypeStruct((E, 128), jnp.int32),
    ]
    blk = lambda b: (b, 0, 0)
    out_specs = [
        pl.BlockSpec((1, 1, TBLK), blk),
        pl.BlockSpec((1, 1, TBLK), blk),
        pl.BlockSpec((1, 1, TBLK), blk),
        pl.BlockSpec((1, 1, TBLK), blk),
        pl.BlockSpec((E, 128), lambda b: (0, 0)),
    ]
    slot0, slot1, w0, w1, counts = pl.pallas_call(
        body,
        grid=(nblk,),
        in_specs=[
            pl.BlockSpec((TBLK, D_MODEL), lambda b: (b, 0)),
            pl.BlockSpec((E, D_MODEL), lambda b: (0, 0)),
        ],
        out_specs=out_specs,
        out_shape=out_shapes,
        scratch_shapes=[pltpu.VMEM((8, 128), jnp.int32)],
    )(x_flat, Wr)
    return (slot0.reshape(T), slot1.reshape(T), w0.reshape(T),
            w1.reshape(T), counts[0, :E])


NW = 32
DISP_CHT = 32


def _sc_dispatch(x_flat, scat_idx, n_slots):
    T, D = x_flat.shape
    tpw = T // NW
    nch = tpw // DISP_CHT
    mesh = plsc.VectorSubcoreMesh(core_axis_name="c", subcore_axis_name="s")

    @functools.partial(
        pl.kernel, mesh=mesh,
        out_type=jax.ShapeDtypeStruct((n_slots, D), jnp.float32),
        scratch_types=[
            pltpu.VMEM((DISP_CHT, D), jnp.float32),
            pltpu.VMEM((DISP_CHT, D), jnp.float32),
            pltpu.VMEM((DISP_CHT,), jnp.int32),
            pltpu.VMEM((DISP_CHT,), jnp.int32),
            pltpu.VMEM((DISP_CHT,), jnp.int32),
            pltpu.VMEM((DISP_CHT,), jnp.int32),
            pltpu.SemaphoreType.DMA,
            pltpu.SemaphoreType.DMA,
            pltpu.SemaphoreType.DMA,
            pltpu.SemaphoreType.DMA,
        ],
    )
    def k(x_hbm, idx_hbm, xe_hbm, xba, xbb, i0a, i1a, i0b, i1b,
          lsa, lsb, ssa, ssb):
        wid = lax.axis_index("s") * 2 + lax.axis_index("c")
        xbufs = (xba, xbb)
        idxs = ((i0a, i1a), (i0b, i1b))
        lsems = (lsa, lsb)
        ssems = (ssa, ssb)

        def start_load(ch):
            tok0 = wid * tpw + ch * DISP_CHT
            return pltpu.async_copy(
                x_hbm.at[pl.ds(tok0, DISP_CHT)], xbufs[ch % 2], lsems[ch % 2])

        loads = {0: start_load(0)}
        scats = {}
        for ch in range(nch):
            if ch + 1 < nch:
                if ch - 1 >= 0:
                    for cp in scats.pop(ch - 1):
                        cp.wait()
                loads[ch + 1] = start_load(ch + 1)
            loads.pop(ch).wait()
            i0, i1 = idxs[ch % 2]
            pltpu.sync_copy(idx_hbm.at[wid, 2 * ch + 0], i0)
            pltpu.sync_copy(idx_hbm.at[wid, 2 * ch + 1], i1)
            scats[ch] = (
                pltpu.async_copy(xbufs[ch % 2], xe_hbm.at[i0], ssems[ch % 2]),
                pltpu.async_copy(xbufs[ch % 2], xe_hbm.at[i1], ssems[ch % 2]))
        for ch in sorted(scats):
            for cp in scats[ch]:
                cp.wait()

    return k(x_flat, scat_idx)


COMB_CHT = 16


def _sc_combine(ye, comb_idx, wb):
    n_slots, D = ye.shape
    T = NW * comb_idx.shape[1] // 2 * COMB_CHT
    tpw = T // NW
    nch = tpw // COMB_CHT
    nl = D // 16
    mesh = plsc.VectorSubcoreMesh(core_axis_name="c", subcore_axis_name="s")

    @functools.partial(
        pl.kernel, mesh=mesh,
        out_type=jax.ShapeDtypeStruct((T, D), jnp.float32),
        scratch_types=[
            pltpu.VMEM((COMB_CHT, D), jnp.float32),
            pltpu.VMEM((COMB_CHT, D), jnp.float32),
            pltpu.VMEM((COMB_CHT, D), jnp.float32),
            pltpu.VMEM((COMB_CHT, D), jnp.float32),
            pltpu.VMEM((COMB_CHT, D), jnp.float32),
            pltpu.VMEM((COMB_CHT,), jnp.int32),
            pltpu.VMEM((COMB_CHT,), jnp.int32),
            pltpu.VMEM((COMB_CHT,), jnp.int32),
            pltpu.VMEM((COMB_CHT,), jnp.int32),
            pltpu.VMEM((COMB_CHT, 16), jnp.float32),
            pltpu.VMEM((COMB_CHT, 16), jnp.float32),
            pltpu.VMEM((COMB_CHT, 16), jnp.float32),
            pltpu.VMEM((COMB_CHT, 16), jnp.float32),
            pltpu.SemaphoreType.DMA,
            pltpu.SemaphoreType.DMA,
            pltpu.SemaphoreType.DMA,
            pltpu.SemaphoreType.DMA,
        ],
    )
    def k(ye_hbm, idx_hbm, wb_hbm, out_hbm,
          r0a, r1a, r0b, r1b, ob, i0a, i1a, i0b, i1b,
          wv0a, wv1a, wv0b, wv1b, s0a, s1a, s0b, s1b):
        wid = lax.axis_index("s") * 2 + lax.axis_index("c")
        rbufs = ((r0a, r1a), (r0b, r1b))
        idxs = ((i0a, i1a), (i0b, i1b))
        wvs = ((wv0a, wv1a), (wv0b, wv1b))
        sems = ((s0a, s1a), (s0b, s1b))

        def start(ch):
            sl = ch % 2
            i0, i1 = idxs[sl]
            wv0, wv1 = wvs[sl]
            pltpu.sync_copy(idx_hbm.at[wid, 2 * ch + 0], i0)
            pltpu.sync_copy(idx_hbm.at[wid, 2 * ch + 1], i1)
            pltpu.sync_copy(wb_hbm.at[wid, 2 * ch + 0], wv0)
            pltpu.sync_copy(wb_hbm.at[wid, 2 * ch + 1], wv1)
            return (pltpu.async_copy(ye_hbm.at[i0], rbufs[sl][0], sems[sl][0]),
                    pltpu.async_copy(ye_hbm.at[i1], rbufs[sl][1], sems[sl][1]))

        pending = {0: start(0)}
        for ch in range(nch):
            if ch + 1 < nch:
                pending[ch + 1] = start(ch + 1)
            for cp in pending.pop(ch):
                cp.wait()
            sl = ch % 2
            r0, r1 = rbufs[sl]
            wv0, wv1 = wvs[sl]

            def body(j, carry):
                a = wv0[j]
                b = wv1[j]
                for l in range(nl):
                    ob[j, pl.ds(l * 16, 16)] = (
                        a * r0[j, pl.ds(l * 16, 16)]
                        + b * r1[j, pl.ds(l * 16, 16)])
                return carry

            lax.fori_loop(0, COMB_CHT, body, 0)
            tok0 = wid * tpw + ch * COMB_CHT
            pltpu.sync_copy(ob, out_hbm.at[pl.ds(tok0, COMB_CHT)])

    return k(ye, comb_idx, wb)


def _ffn_body(counts_ref, xe_ref, w1_ref, w2_ref, out_ref, acc_ref, xb_ref):
    f = pl.program_id(1)
    e = pl.program_id(0)

    @pl.when(f == 0)
    def _init():
        xb_ref[...] = xe_ref[...].astype(jnp.bfloat16)

    w1b = w1_ref[0].astype(jnp.bfloat16)
    w2b = w2_ref[0].astype(jnp.bfloat16)
    cnt = counts_ref[e]
    xb = xb_ref[...]
    h = jax.lax.dot_general(
        xb, w1b, (((1,), (1,)), ((), ())),
        preferred_element_type=jnp.float32)
    h = 0.5 * h * (1.0 + jax.lax.erf(h * np.float32(0.7071067811865476)))
    contrib = jax.lax.dot_general(
        h.astype(jnp.bfloat16), w2b, (((1,), (1,)), ((), ())),
        preferred_element_type=jnp.float32)

    @pl.when(f == 0)
    def _first():
        acc_ref[...] = contrib

    @pl.when(f != 0)
    def _rest():
        acc_ref[...] += contrib

    @pl.when(f == NF - 1)
    def _fin():
        rows = jax.lax.broadcasted_iota(jnp.int32, acc_ref.shape, 0)
        out_ref[...] = jnp.where(rows < cnt, acc_ref[...], 0.0)


def _expert_ffn(xe, W1, W2, counts, cap):
    return pl.pallas_call(
        _ffn_body,
        grid=(E, NF),
        in_specs=[
            pl.BlockSpec(memory_space=pltpu.SMEM),
            pl.BlockSpec((cap, D_MODEL), lambda e, f: (e, 0)),
            pl.BlockSpec((1, FBLK, D_MODEL), lambda e, f: (e, f, 0)),
            pl.BlockSpec((1, D_MODEL, FBLK), lambda e, f: (e, 0, f)),
        ],
        out_specs=pl.BlockSpec((cap, D_MODEL), lambda e, f: (e, 0)),
        out_shape=jax.ShapeDtypeStruct((E * cap, D_MODEL), jnp.float32),
        scratch_shapes=[
            pltpu.VMEM((cap, D_MODEL), jnp.float32),
            pltpu.VMEM((cap, D_MODEL), jnp.bfloat16),
        ],
    )(counts, xe, W1, W2)


def kernel(x, Wr, W1, W2):
    B, S, D = x.shape
    x_flat = x.reshape(-1, D)
    T = x_flat.shape[0]
    cap = max(1, int(np.ceil(T * K * CAP_FACTOR / E)))

    slot0, slot1, w0, w1, counts_k = _route(x_flat, Wr, cap)

    tpw = T // NW
    nch = tpw // DISP_CHT
    scat_idx = (
        jnp.stack([slot0, slot1], axis=1)
        .reshape(NW, nch, DISP_CHT, 2)
        .transpose(0, 1, 3, 2)
        .reshape(NW, nch * 2, DISP_CHT))
    xe_buf = _sc_dispatch(x_flat, scat_idx, E * cap + 8)

    ye = _expert_ffn(xe_buf, W1, W2, counts_k, cap)

    nchc = (T // NW) // COMB_CHT
    comb_idx = (
        jnp.minimum(jnp.stack([slot0, slot1], axis=1), E * cap - 1)
        .reshape(NW, nchc, COMB_CHT, 2)
        .transpose(0, 1, 3, 2)
        .reshape(NW, nchc * 2, COMB_CHT))
    wb = (
        jnp.stack([w0, w1], axis=1)
        .reshape(NW, nchc, COMB_CHT, 2)
        .transpose(0, 1, 3, 2)
        .reshape(NW, nchc * 2, COMB_CHT)[..., None]
        * jnp.ones((1, 1, 1, 16), jnp.float32))
    out = _sc_combine(ye, comb_idx, wb)
    return out.reshape(B, S, D)
